# baseline (speedup 1.0000x reference)
import functools

import jax
import jax.numpy as jnp
from jax import lax
from jax.experimental import pallas as pl
from jax.experimental.pallas import tpu as pltpu
from jax.experimental.pallas import tpu_sc as plsc

_NUM_CORES = 2
_NUM_SUBCORES = 16
_LANES = 16
_NW = _NUM_CORES * _NUM_SUBCORES

_VOC_PAD = 256


def _histogram_sc(pos_ids, dep_ids, B, S, dep_offset):
    rows_per_w = B // _NW
    SUPER = 128
    n_super = rows_per_w // SUPER
    n_sub = SUPER // _LANES

    mesh = plsc.VectorSubcoreMesh(core_axis_name="c", subcore_axis_name="s")

    @functools.partial(
        pl.kernel,
        out_type=jax.ShapeDtypeStruct((2 * B, 128), jnp.float32),
        mesh=mesh,
        compiler_params=pltpu.CompilerParams(use_tc_tiling_on_sc=True,
                                             needs_layout_passes=False),
        scratch_types=[
            pltpu.VMEM((SUPER, S), jnp.int32),
            pltpu.VMEM((SUPER, S), jnp.int32),
            pltpu.VMEM((2 * SUPER, 128), jnp.float32),
        ],
    )
    def hist(pos_hbm, dep_hbm, out_hbm, pos_v, dep_v, cnt_v):
        wid = lax.axis_index("s") * _NUM_CORES + lax.axis_index("c")
        iota = lax.iota(jnp.int32, _LANES)
        ones = jnp.ones((_LANES,), jnp.float32)
        zeros = jnp.zeros((_LANES,), jnp.float32)

        UNROLL = 8
        assert S % UNROLL == 0

        def super_body(c, carry):
            base = wid * rows_per_w + c * SUPER
            pltpu.sync_copy(pos_hbm.at[pl.ds(base, SUPER), :], pos_v)
            pltpu.sync_copy(dep_hbm.at[pl.ds(base, SUPER), :], dep_v)

            for i in range(2 * SUPER):
                for j in range(128 // _LANES):
                    cnt_v[i, pl.ds(j * _LANES, _LANES)] = zeros

            for sub in range(n_sub):
                rowv = iota + sub * _LANES

                @plsc.parallel_loop(0, S, step=1, unroll=UNROLL)
                def s_body(s):
                    scol = jnp.broadcast_to(s, (_LANES,)).astype(
                        jnp.int32) + iota
                    scol = jnp.where(scol >= S, scol - S, scol)
                    p = plsc.load_gather(pos_v, [rowv, scol])
                    plsc.addupdate_scatter(cnt_v, [rowv, p], ones)
                    d = plsc.load_gather(dep_v, [rowv, scol]) + dep_offset
                    drow = rowv + jnp.bitwise_and(d, -128)
                    dcol = jnp.bitwise_and(d, 127)
                    plsc.addupdate_scatter(cnt_v, [drow, dcol], ones)

            pltpu.sync_copy(cnt_v.at[pl.ds(0, SUPER), :],
                            out_hbm.at[pl.ds(base, SUPER), :])
            pltpu.sync_copy(cnt_v.at[pl.ds(SUPER, SUPER), :],
                            out_hbm.at[pl.ds(B + base, SUPER), :])
            return carry

        lax.fori_loop(0, n_super, super_body, 0)

    return hist(pos_ids, dep_ids)


def _finish_tc(counts2, T_lo, T_hi, W, b2, inv_s, B):
    OD = W.shape[0]
    BT = 1024
    nb = B // BT

    def body(lo_ref, hi_ref, tlo_ref, thi_ref, w_ref, b_ref, o_ref):
        comb = (jnp.dot(lo_ref[...], tlo_ref[...],
                        preferred_element_type=jnp.float32) +
                jnp.dot(hi_ref[...], thi_ref[...],
                        preferred_element_type=jnp.float32)) * inv_s
        out = lax.dot_general(comb, w_ref[...],
                              dimension_numbers=(((1,), (1,)), ((), ())),
                              preferred_element_type=jnp.float32)
        o_ref[...] = jnp.maximum(out + b_ref[...], 0.0)

    return pl.pallas_call(
        body,
        grid=(nb,),
        in_specs=[
            pl.BlockSpec((BT, 128), lambda i: (i, 0)),
            pl.BlockSpec((BT, 128), lambda i: (i + nb, 0)),
            pl.BlockSpec((128, 128), lambda i: (0, 0)),
            pl.BlockSpec((128, 128), lambda i: (0, 0)),
            pl.BlockSpec((OD, 128), lambda i: (0, 0)),
            pl.BlockSpec((1, OD), lambda i: (0, 0)),
        ],
        out_specs=pl.BlockSpec((BT, OD), lambda i: (i, 0)),
        out_shape=jax.ShapeDtypeStruct((B, OD), jnp.float32),
    )(counts2, counts2, T_lo, T_hi, W, b2)


def kernel(pos_ids, dep_ids, pos_table, dep_table, W, b):
    B, S = pos_ids.shape
    NP, ED = pos_table.shape
    ND = dep_table.shape[1 - 1]

    T_cat = jnp.zeros((_VOC_PAD, 2 * ED), jnp.float32)
    T_cat = T_cat.at[0:NP, 0:ED].set(pos_table)
    T_cat = T_cat.at[NP:NP + ND, ED:2 * ED].set(dep_table)
    T_lo = T_cat[0:128]
    T_hi = T_cat[128:256]

    W = W.astype(jnp.float32)
    b2 = b.astype(jnp.float32).reshape(1, -1)
    pos_ids = pos_ids.astype(jnp.int32)
    dep_ids = dep_ids.astype(jnp.int32)

    NSLICE = 2
    BS = B // NSLICE
    outs = []
    for i in range(NSLICE):
        cnt = _histogram_sc(
            lax.slice_in_dim(pos_ids, i * BS, (i + 1) * BS, axis=0),
            lax.slice_in_dim(dep_ids, i * BS, (i + 1) * BS, axis=0),
            BS, S, NP)
        outs.append(_finish_tc(cnt, T_lo, T_hi, W, b2, 1.0 / S, BS))
    return jnp.concatenate(outs, axis=0)


# speedup vs baseline: 1.2099x; 1.2099x over previous
import functools

import jax
import jax.numpy as jnp
from jax import lax
from jax.experimental import pallas as pl
from jax.experimental.pallas import tpu as pltpu
from jax.experimental.pallas import tpu_sc as plsc

_NUM_CORES = 2
_NUM_SUBCORES = 16
_LANES = 16
_NW = _NUM_CORES * _NUM_SUBCORES

_VOC_PAD = 256


def _histogram_sc(pos_ids, dep_ids, B, S, dep_offset):
    rows_per_w = B // _NW
    SUPER = 128
    n_super = rows_per_w // SUPER
    n_sub = SUPER // _LANES

    mesh = plsc.VectorSubcoreMesh(core_axis_name="c", subcore_axis_name="s")

    @functools.partial(
        pl.kernel,
        out_type=jax.ShapeDtypeStruct((2 * B, 128), jnp.float32),
        mesh=mesh,
        compiler_params=pltpu.CompilerParams(use_tc_tiling_on_sc=True,
                                             needs_layout_passes=False),
        scratch_types=[
            pltpu.VMEM((SUPER, S), jnp.int32),
            pltpu.VMEM((SUPER, S), jnp.int32),
            pltpu.VMEM((2 * SUPER, 128), jnp.float32),
        ],
    )
    def hist(pos_hbm, dep_hbm, out_hbm, pos_v, dep_v, cnt_v):
        wid = lax.axis_index("s") * _NUM_CORES + lax.axis_index("c")
        iota = lax.iota(jnp.int32, _LANES)
        ones = jnp.ones((_LANES,), jnp.float32)
        zeros = jnp.zeros((_LANES,), jnp.float32)

        UNROLL = 8
        assert S % UNROLL == 0

        def super_body(c, carry):
            base = wid * rows_per_w + c * SUPER
            pltpu.sync_copy(pos_hbm.at[pl.ds(base, SUPER), :], pos_v)
            pltpu.sync_copy(dep_hbm.at[pl.ds(base, SUPER), :], dep_v)

            for i in range(2 * SUPER):
                for j in range(128 // _LANES):
                    cnt_v[i, pl.ds(j * _LANES, _LANES)] = zeros

            for sub in range(n_sub):
                rowv = iota + sub * _LANES

                @plsc.parallel_loop(0, S, step=1, unroll=UNROLL)
                def s_body(s):
                    scol = jnp.broadcast_to(s, (_LANES,)).astype(
                        jnp.int32) + iota
                    scol = jnp.where(scol >= S, scol - S, scol)
                    p = plsc.load_gather(pos_v, [rowv, scol])
                    plsc.addupdate_scatter(cnt_v, [rowv, p], ones)
                    d = plsc.load_gather(dep_v, [rowv, scol]) + dep_offset
                    drow = rowv + jnp.bitwise_and(d, -128)
                    dcol = jnp.bitwise_and(d, 127)
                    plsc.addupdate_scatter(cnt_v, [drow, dcol], ones)

            pltpu.sync_copy(cnt_v.at[pl.ds(0, SUPER), :],
                            out_hbm.at[pl.ds(base, SUPER), :])
            pltpu.sync_copy(cnt_v.at[pl.ds(SUPER, SUPER), :],
                            out_hbm.at[pl.ds(B + base, SUPER), :])
            return carry

        lax.fori_loop(0, n_super, super_body, 0)

    return hist(pos_ids, dep_ids)


def _finish_tc(counts2, T_lo, T_hi, W, b2, inv_s, B):
    OD = W.shape[0]
    BT = 1024
    nb = B // BT

    def body(lo_ref, hi_ref, tlo_ref, thi_ref, w_ref, b_ref, o_ref):
        comb = (jnp.dot(lo_ref[...], tlo_ref[...],
                        preferred_element_type=jnp.float32) +
                jnp.dot(hi_ref[...], thi_ref[...],
                        preferred_element_type=jnp.float32)) * inv_s
        out = lax.dot_general(comb, w_ref[...],
                              dimension_numbers=(((1,), (1,)), ((), ())),
                              preferred_element_type=jnp.float32)
        o_ref[...] = jnp.maximum(out + b_ref[...], 0.0)

    return pl.pallas_call(
        body,
        grid=(nb,),
        in_specs=[
            pl.BlockSpec((BT, 128), lambda i: (i, 0)),
            pl.BlockSpec((BT, 128), lambda i: (i + nb, 0)),
            pl.BlockSpec((128, 128), lambda i: (0, 0)),
            pl.BlockSpec((128, 128), lambda i: (0, 0)),
            pl.BlockSpec((OD, 128), lambda i: (0, 0)),
            pl.BlockSpec((1, OD), lambda i: (0, 0)),
        ],
        out_specs=pl.BlockSpec((BT, OD), lambda i: (i, 0)),
        out_shape=jax.ShapeDtypeStruct((B, OD), jnp.float32),
    )(counts2, counts2, T_lo, T_hi, W, b2)


def kernel(pos_ids, dep_ids, pos_table, dep_table, W, b):
    B, S = pos_ids.shape
    NP, ED = pos_table.shape
    ND = dep_table.shape[1 - 1]

    T_cat = jnp.zeros((_VOC_PAD, 2 * ED), jnp.float32)
    T_cat = T_cat.at[0:NP, 0:ED].set(pos_table)
    T_cat = T_cat.at[NP:NP + ND, ED:2 * ED].set(dep_table)
    T_lo = T_cat[0:128]
    T_hi = T_cat[128:256]

    counts2 = _histogram_sc(pos_ids.astype(jnp.int32),
                            dep_ids.astype(jnp.int32), B, S, NP)

    return _finish_tc(counts2, T_lo, T_hi, W.astype(jnp.float32),
                      b.astype(jnp.float32).reshape(1, -1), 1.0 / S, B)
